# baseline (speedup 1.0000x reference)
import functools

import jax
import jax.numpy as jnp
from jax import lax
from jax.experimental import pallas as pl
from jax.experimental.pallas import tpu as pltpu
from jax.experimental.pallas import tpu_sc as plsc

N_FEAT = 64
N_WORDS = 100000
BATCH_N = 16384
NC, NS = 2, 16
LANES = 16
CHUNKP = 4096
NCHUNKP = BATCH_N // CHUNKP
ROWS16 = BATCH_N // LANES
F_PER_W = N_FEAT // (NC * NS)


@jax.jit
def _sc_feature_dot(Et, W, batch_t):
    mesh = plsc.VectorSubcoreMesh(
        core_axis_name="c", subcore_axis_name="s",
        num_cores=NC, num_subcores=NS,
    )

    @functools.partial(
        pl.kernel,
        out_type=jax.ShapeDtypeStruct((NC, ROWS16, LANES), jnp.float32),
        mesh=mesh,
        scratch_types=[
            pltpu.VMEM((N_WORDS,), jnp.float32),
            pltpu.VMEM((BATCH_N,), jnp.float32),
            pltpu.VMEM((CHUNKP,), jnp.int32),
            pltpu.VMEM((CHUNKP // LANES, LANES), jnp.float32),
            pltpu.VMEM((NCHUNKP, ROWS16 // NCHUNKP), jnp.int32),
            pltpu.VMEM_SHARED((ROWS16, LANES), jnp.float32),
            pltpu.SemaphoreType.DMA,
        ],
        compiler_params=pltpu.CompilerParams(
            needs_layout_passes=False, use_tc_tiling_on_sc=False),
    )
    def k(et_hbm, w_hbm, b_hbm, out_hbm,
          row_v, wv_v, idx_v, stage_v, ramp_v, acc_sh, sem):
        c = lax.axis_index("c")
        s = lax.axis_index("s")

        nrow = ROWS16 // NCHUNKP
        for q in range(NCHUNKP):
            def ramp_body(g, carry, q=q):
                ramp_v.at[q][pl.ds(g * LANES, LANES)] = (
                    lax.iota(jnp.int32, LANES) + (q * nrow + g * LANES))
                return carry
            lax.fori_loop(0, nrow // LANES, ramp_body, 0, unroll=True)

        @pl.when(s == 0)
        def _():
            def z_body(g, carry):
                stage_v[g, pl.ds(0, LANES)] = jnp.zeros((LANES,), jnp.float32)
                return carry
            lax.fori_loop(0, CHUNKP // LANES, z_body, 0, unroll=True)
            for q in range(NCHUNKP):
                pltpu.sync_copy(
                    stage_v, acc_sh.at[pl.ds(q * (CHUNKP // LANES), CHUNKP // LANES)])
        plsc.subcore_barrier()

        for fk in range(F_PER_W):
            f = c * (NS * F_PER_W) + s * F_PER_W + fk

            pltpu.sync_copy(w_hbm.at[f], row_v)
            for q in range(NCHUNKP):
                pltpu.sync_copy(b_hbm.at[1, pl.ds(q * CHUNKP, CHUNKP)], idx_v)

                def wg_body(g, carry):
                    idx16 = idx_v[pl.ds(g * LANES, LANES)]
                    wv_v[pl.ds(q * CHUNKP + g * LANES, LANES)] = \
                        plsc.load_gather(row_v, [idx16])
                    return carry
                lax.fori_loop(0, CHUNKP // LANES, wg_body, 0, unroll=8)

            pltpu.sync_copy(et_hbm.at[f, pl.ds(0, N_WORDS)], row_v)
            for q in range(NCHUNKP):
                pltpu.sync_copy(b_hbm.at[0, pl.ds(q * CHUNKP, CHUNKP)], idx_v)

                def eg_body(g, carry):
                    idx16 = idx_v[pl.ds(g * LANES, LANES)]
                    e16 = plsc.load_gather(row_v, [idx16])
                    w16 = wv_v[pl.ds(q * CHUNKP + g * LANES, LANES)]
                    stage_v[g, pl.ds(0, LANES)] = e16 * w16
                    return carry
                lax.fori_loop(0, CHUNKP // LANES, eg_body, 0, unroll=8)

                pltpu.sync_copy(
                    stage_v,
                    acc_sh.at[ramp_v.at[q]],
                    add=True,
                )

        plsc.subcore_barrier()

        @pl.when(s == 0)
        def _():
            pltpu.sync_copy(acc_sh, out_hbm.at[c])

    return k(Et, W, batch_t)


def _combine_block(p_ref, o_ref):
    o_ref[...] = p_ref[0] + p_ref[1]


def _tc_combine(partial):
    p3 = partial.reshape(NC, 128, 128)
    out = pl.pallas_call(
        _combine_block,
        in_specs=[pl.BlockSpec((NC, 128, 128), lambda: (0, 0, 0))],
        out_specs=pl.BlockSpec((128, 128), lambda: (0, 0)),
        out_shape=jax.ShapeDtypeStruct((128, 128), jnp.float32),
    )(p3)
    return out.reshape(BATCH_N)


def kernel(batch, E, W):
    Et = E.T
    bt = batch.astype(jnp.int32).T
    partial = _sc_feature_dot(Et, W, bt)
    return _tc_combine(partial)
